# baseline (speedup 1.0000x reference)
import functools

import jax
import jax.numpy as jnp
from jax import lax
from jax.experimental import pallas as pl
from jax.experimental.pallas import tpu as pltpu
from jax.experimental.pallas import tpu_sc as plsc

NUM_FIELDS = 26
VOCAB = 100000
EMB_DIM = 32
BATCH = 16384

NC = 2
NS = 16
NW = NC * NS
BPW = BATCH // NW
CB = 64
NCHUNK = BPW // CB
ROWS_PER_CHUNK = CB * NUM_FIELDS
IDX_W = 128
GPC = ROWS_PER_CHUNK // IDX_W
IDX_ROWS = BPW * NUM_FIELDS // IDX_W


def _sc_body(idx_hbm, table_hbm, out_hbm, idx_v, buf_v, outb_v, sem):
    c = lax.axis_index("c")
    s = lax.axis_index("s")
    wid = s * NC + c
    base = wid * BPW

    pltpu.sync_copy(idx_hbm.at[wid], idx_v)

    def chunk_body(ch, carry):
        copies = []
        for j in range(GPC):
            cp = pltpu.async_copy(
                table_hbm.at[idx_v.at[ch * GPC + j]],
                buf_v.at[pl.ds(j * IDX_W, IDX_W)],
                sem,
            )
            copies.append(cp)
        for cp in copies:
            cp.wait()

        def red_body(j, carry2):
            r = j * NUM_FIELDS
            a0 = buf_v[r, pl.ds(0, 16)]
            a1 = buf_v[r, pl.ds(16, 16)]
            for f in range(1, NUM_FIELDS):
                a0 = a0 + buf_v[r + f, pl.ds(0, 16)]
                a1 = a1 + buf_v[r + f, pl.ds(16, 16)]
            outb_v[j, pl.ds(0, 16)] = a0
            outb_v[j, pl.ds(16, 16)] = a1
            return carry2

        lax.fori_loop(0, CB, red_body, 0)

        pltpu.sync_copy(outb_v, out_hbm.at[pl.ds(base + ch * CB, CB)])
        return carry

    lax.fori_loop(0, NCHUNK, chunk_body, 0)


_emb_call = functools.partial(
    pl.kernel,
    mesh=plsc.VectorSubcoreMesh(
        core_axis_name="c", subcore_axis_name="s", num_cores=NC, num_subcores=NS
    ),
    out_type=jax.ShapeDtypeStruct((BATCH, EMB_DIM), jnp.float32),
    scratch_types=[
        pltpu.VMEM((IDX_ROWS, IDX_W), jnp.int32),
        pltpu.VMEM((ROWS_PER_CHUNK, EMB_DIM), jnp.float32),
        pltpu.VMEM((CB, EMB_DIM), jnp.float32),
        pltpu.SemaphoreType.DMA,
    ],
    compiler_params=pltpu.CompilerParams(use_tc_tiling_on_sc=False),
)(_sc_body)


@jax.jit
def kernel(g, x, tables):
    x = x.astype(jnp.int32)
    offs = (jnp.arange(NUM_FIELDS, dtype=jnp.int32) * VOCAB)[None, :]
    flat_idx = (x + offs).reshape(NW, IDX_ROWS, IDX_W)
    table = tables.reshape(NUM_FIELDS * VOCAB, EMB_DIM)
    return _emb_call(flat_idx, table)


# speedup vs baseline: 1.0145x; 1.0145x over previous
import functools

import jax
import jax.numpy as jnp
from jax import lax
from jax.experimental import pallas as pl
from jax.experimental.pallas import tpu as pltpu
from jax.experimental.pallas import tpu_sc as plsc

NUM_FIELDS = 26
VOCAB = 100000
EMB_DIM = 32
BATCH = 16384

NC = 2
NS = 16
NW = NC * NS
BPW = BATCH // NW
CB = 64
NCHUNK = BPW // CB
ROWS_PER_CHUNK = CB * NUM_FIELDS
IDX_W = 128
GPC = ROWS_PER_CHUNK // IDX_W


def _sc_body(idx_hbm, table_hbm, out_hbm, idx_v, buf_v, outb_v, sem):
    c = lax.axis_index("c")
    s = lax.axis_index("s")
    wid = s * NC + c
    base = wid * BPW

    pltpu.sync_copy(idx_hbm.at[wid], idx_v)

    def start_gather(ch, slot):
        return pltpu.async_copy(
            table_hbm.at[idx_v.at[ch]], buf_v.at[slot], sem
        )

    start_gather(0, 0)

    def chunk_body(ch, carry):
        slot = lax.rem(ch, 2)
        nslot = lax.rem(ch + 1, 2)

        @pl.when(ch + 1 < NCHUNK)
        def _():
            start_gather(ch + 1, nslot)

        pltpu.make_async_copy(
            table_hbm.at[idx_v.at[ch]], buf_v.at[slot], sem
        ).wait()

        def red_body(j, carry2):
            a0 = buf_v[slot, j, pl.ds(0, 16)]
            a1 = buf_v[slot, j, pl.ds(16, 16)]
            for f in range(1, NUM_FIELDS):
                row = f * CB + j
                a0 = a0 + buf_v[slot, row, pl.ds(0, 16)]
                a1 = a1 + buf_v[slot, row, pl.ds(16, 16)]
            outb_v[j, pl.ds(0, 16)] = a0
            outb_v[j, pl.ds(16, 16)] = a1
            return carry2

        lax.fori_loop(0, CB, red_body, 0)

        pltpu.sync_copy(outb_v, out_hbm.at[pl.ds(base + ch * CB, CB)])
        return carry

    lax.fori_loop(0, NCHUNK, chunk_body, 0)


_emb_call = functools.partial(
    pl.kernel,
    mesh=plsc.VectorSubcoreMesh(
        core_axis_name="c", subcore_axis_name="s", num_cores=NC, num_subcores=NS
    ),
    out_type=jax.ShapeDtypeStruct((BATCH, EMB_DIM), jnp.float32),
    scratch_types=[
        pltpu.VMEM((NCHUNK, ROWS_PER_CHUNK), jnp.int32),
        pltpu.VMEM((2, ROWS_PER_CHUNK, EMB_DIM), jnp.float32),
        pltpu.VMEM((CB, EMB_DIM), jnp.float32),
        pltpu.SemaphoreType.DMA,
    ],
    compiler_params=pltpu.CompilerParams(use_tc_tiling_on_sc=False),
)(_sc_body)


@jax.jit
def kernel(g, x, tables):
    x = x.astype(jnp.int32)
    offs = (jnp.arange(NUM_FIELDS, dtype=jnp.int32) * VOCAB)[None, :]
    flat = x + offs
    flat = flat.reshape(NW, NCHUNK, CB, NUM_FIELDS).transpose(0, 1, 3, 2)
    idx = flat.reshape(NW, NCHUNK, ROWS_PER_CHUNK)
    table = tables.reshape(NUM_FIELDS * VOCAB, EMB_DIM)
    return _emb_call(idx, table)
